# Initial kernel scaffold; baseline (speedup 1.0000x reference)
import jax
import jax.numpy as jnp
from jax.experimental import pallas as pl


def kernel(x, Wg_w, Wg_b, Wn_w, Wn_b):
    raise NotImplementedError("write your pallas kernel here")


# speedup vs baseline: 1.8478x; 1.8478x over previous
import functools

import jax
import jax.numpy as jnp
from jax.experimental import pallas as pl

TOKENS = 8192
M = 2048
T = 64
K = 2
BLK = 512


def _router_block(x_ref, w_ref, b_ref, noise_ref,
                  gates_ref, h_ref, idx_ref, ns_ref, logits_ref):
    xb = x_ref[...]
    w = w_ref[...]
    acc = jnp.dot(xb.astype(jnp.bfloat16), w.astype(jnp.bfloat16),
                  preferred_element_type=jnp.float32)
    b = b_ref[...]
    acc = acc + b
    logits = acc[:, :T]
    pre = acc[:, T:]
    ns = jnp.maximum(pre, 0.0) + jnp.log1p(jnp.exp(-jnp.abs(pre)))
    h = logits + noise_ref[...] * ns

    iota = jax.lax.broadcasted_iota(jnp.int32, (BLK, T), 1)
    v1 = jnp.max(h, axis=-1, keepdims=True)
    i1 = jnp.min(jnp.where(h == v1, iota, T), axis=-1, keepdims=True)
    h2 = jnp.where(iota == i1, -jnp.inf, h)
    v2 = jnp.max(h2, axis=-1, keepdims=True)
    i2 = jnp.min(jnp.where(h2 == v2, iota, T), axis=-1, keepdims=True)

    e2 = jnp.exp(v2 - v1)
    denom = 1.0 + e2
    p1 = 1.0 / denom
    p2 = e2 / denom
    gates = jnp.where(iota == i1, p1, jnp.where(iota == i2, p2, 0.0))

    gates_ref[...] = gates
    h_ref[...] = h
    idx_ref[...] = jnp.concatenate([i1, i2], axis=1)
    ns_ref[...] = ns
    logits_ref[...] = logits


@functools.partial(jax.jit, static_argnums=())
def kernel(x, Wg_w, Wg_b, Wn_w, Wn_b):
    w = jnp.concatenate([Wg_w, Wn_w], axis=0).T
    b = jnp.concatenate([Wg_b, Wn_b], axis=0)[None, :]
    noise = jax.random.normal(jax.random.key(42), (TOKENS, T),
                              dtype=jnp.float32)
    grid = (TOKENS // BLK,)
    out = pl.pallas_call(
        _router_block,
        grid=grid,
        in_specs=[
            pl.BlockSpec((BLK, M), lambda i: (i, 0)),
            pl.BlockSpec((M, 2 * T), lambda i: (0, 0)),
            pl.BlockSpec((1, 2 * T), lambda i: (0, 0)),
            pl.BlockSpec((BLK, T), lambda i: (i, 0)),
        ],
        out_specs=[
            pl.BlockSpec((BLK, T), lambda i: (i, 0)),
            pl.BlockSpec((BLK, T), lambda i: (i, 0)),
            pl.BlockSpec((BLK, K), lambda i: (i, 0)),
            pl.BlockSpec((BLK, T), lambda i: (i, 0)),
            pl.BlockSpec((BLK, T), lambda i: (i, 0)),
        ],
        out_shape=[
            jax.ShapeDtypeStruct((TOKENS, T), jnp.float32),
            jax.ShapeDtypeStruct((TOKENS, T), jnp.float32),
            jax.ShapeDtypeStruct((TOKENS, K), jnp.int32),
            jax.ShapeDtypeStruct((TOKENS, T), jnp.float32),
            jax.ShapeDtypeStruct((TOKENS, T), jnp.float32),
        ],
    )(x, w, b, noise)
    gates, h, topk_idx, noise_scale, logits = out
    return (gates, h, topk_idx, noise_scale, logits)


# speedup vs baseline: 1.9348x; 1.0471x over previous
import functools

import jax
import jax.numpy as jnp
from jax.experimental import pallas as pl

TOKENS = 8192
M = 2048
T = 64
K = 2
BLK = 1024


def _router_block(x_ref, w_ref, b_ref, noise_ref,
                  gates_ref, h_ref, idx_ref, ns_ref, logits_ref):
    xb = x_ref[...]
    w = w_ref[...]
    acc = jnp.dot(xb.astype(jnp.bfloat16), w.astype(jnp.bfloat16),
                  preferred_element_type=jnp.float32)
    b = b_ref[...]
    acc = acc + b
    logits = acc[:, :T]
    pre = acc[:, T:]
    ns = jnp.maximum(pre, 0.0) + jnp.log1p(jnp.exp(-jnp.abs(pre)))
    h = logits + noise_ref[...] * ns

    iota = jax.lax.broadcasted_iota(jnp.int32, (BLK, T), 1)
    v1 = jnp.max(h, axis=-1, keepdims=True)
    i1 = jnp.min(jnp.where(h == v1, iota, T), axis=-1, keepdims=True)
    h2 = jnp.where(iota == i1, -jnp.inf, h)
    v2 = jnp.max(h2, axis=-1, keepdims=True)
    i2 = jnp.min(jnp.where(h2 == v2, iota, T), axis=-1, keepdims=True)

    e2 = jnp.exp(v2 - v1)
    denom = 1.0 + e2
    p1 = 1.0 / denom
    p2 = e2 / denom
    gates = jnp.where(iota == i1, p1, jnp.where(iota == i2, p2, 0.0))

    gates_ref[...] = gates
    h_ref[...] = h
    idx_ref[...] = jnp.concatenate([i1, i2], axis=1)
    ns_ref[...] = ns
    logits_ref[...] = logits


@functools.partial(jax.jit, static_argnums=())
def kernel(x, Wg_w, Wg_b, Wn_w, Wn_b):
    w = jnp.concatenate([Wg_w, Wn_w], axis=0).T
    b = jnp.concatenate([Wg_b, Wn_b], axis=0)[None, :]
    noise = jax.random.normal(jax.random.key(42), (TOKENS, T),
                              dtype=jnp.float32)
    grid = (TOKENS // BLK,)
    out = pl.pallas_call(
        _router_block,
        grid=grid,
        in_specs=[
            pl.BlockSpec((BLK, M), lambda i: (i, 0)),
            pl.BlockSpec((M, 2 * T), lambda i: (0, 0)),
            pl.BlockSpec((1, 2 * T), lambda i: (0, 0)),
            pl.BlockSpec((BLK, T), lambda i: (i, 0)),
        ],
        out_specs=[
            pl.BlockSpec((BLK, T), lambda i: (i, 0)),
            pl.BlockSpec((BLK, T), lambda i: (i, 0)),
            pl.BlockSpec((BLK, K), lambda i: (i, 0)),
            pl.BlockSpec((BLK, T), lambda i: (i, 0)),
            pl.BlockSpec((BLK, T), lambda i: (i, 0)),
        ],
        out_shape=[
            jax.ShapeDtypeStruct((TOKENS, T), jnp.float32),
            jax.ShapeDtypeStruct((TOKENS, T), jnp.float32),
            jax.ShapeDtypeStruct((TOKENS, K), jnp.int32),
            jax.ShapeDtypeStruct((TOKENS, T), jnp.float32),
            jax.ShapeDtypeStruct((TOKENS, T), jnp.float32),
        ],
    )(x, w, b, noise)
    gates, h, topk_idx, noise_scale, logits = out
    return (gates, h, topk_idx, noise_scale, logits)
